# Initial kernel scaffold; baseline (speedup 1.0000x reference)
import jax
import jax.numpy as jnp
from jax.experimental import pallas as pl


def kernel(p, x, o, Ws1, gs, bs, Ws2, bs2, W2, g2, b2):
    raise NotImplementedError("write your pallas kernel here")


# speedup vs baseline: 10.0311x; 10.0311x over previous
import functools

import jax
import jax.numpy as jnp
from jax import lax
from jax.experimental import pallas as pl
from jax.experimental.pallas import tpu as pltpu
from jax.experimental.pallas import tpu_sc as plsc

N = 16384
CIN = 64
COUT = 128
M = 4096
K = 16
EPS = 1e-5


def _fps_body(px_ref, py_ref, pz_ref, npx_ref, npy_ref, npz_ref, dist_ref):
    px = px_ref[...]
    py = py_ref[...]
    pz = pz_ref[...]
    row = lax.broadcasted_iota(jnp.int32, (128, 128), 0)
    col = lax.broadcasted_iota(jnp.int32, (128, 128), 1)
    iota = row * 128 + col
    dist_ref[...] = jnp.full((128, 128), 1e10, jnp.float32)
    npx_ref[0:1, :] = px[0:1, 0:1]
    npy_ref[0:1, :] = py[0:1, 0:1]
    npz_ref[0:1, :] = pz[0:1, 0:1]

    def body(i, carry):
        lx, ly, lz = carry
        dx = px - lx
        dy = py - ly
        dz = pz - lz
        d = (dx * dx + dz * dz) + dy * dy
        dist = jnp.minimum(dist_ref[...], d)
        dist_ref[...] = dist
        mx = jnp.max(dist)
        nxt = jnp.min(jnp.where(dist == mx, iota, jnp.int32(N)))
        sel = iota == nxt
        neg = jnp.float32(-3e38)
        nlx = jnp.max(jnp.where(sel, px, neg))
        nly = jnp.max(jnp.where(sel, py, neg))
        nlz = jnp.max(jnp.where(sel, pz, neg))
        npx_ref[pl.ds(i, 1), :] = jnp.full((1, 1), nlx, jnp.float32)
        npy_ref[pl.ds(i, 1), :] = jnp.full((1, 1), nly, jnp.float32)
        npz_ref[pl.ds(i, 1), :] = jnp.full((1, 1), nlz, jnp.float32)
        return (nlx, nly, nlz)

    sel0 = iota == 0
    neg0 = jnp.float32(-3e38)
    lx0 = jnp.max(jnp.where(sel0, px, neg0))
    ly0 = jnp.max(jnp.where(sel0, py, neg0))
    lz0 = jnp.max(jnp.where(sel0, pz, neg0))
    lax.fori_loop(1, M, body, (lx0, ly0, lz0))


def _fps(px, py, pz):
    out = jax.ShapeDtypeStruct((M, 1), jnp.float32)
    return pl.pallas_call(
        _fps_body,
        out_shape=(out, out, out),
        scratch_shapes=[pltpu.VMEM((128, 128), jnp.float32)],
    )(px, py, pz)


_KNN_R = 128


def _knn_body(pxr_ref, pyr_ref, pzr_ref, npx_ref, npy_ref, npz_ref, knn_ref):
    px = pxr_ref[...]
    py = pyr_ref[...]
    pz = pzr_ref[...]
    dx = npx_ref[...] - px
    dy = npy_ref[...] - py
    dz = npz_ref[...] - pz
    d2 = dx * dx + dy * dy + dz * dz
    col = lax.broadcasted_iota(jnp.int32, (_KNN_R, N), 1)
    inf = jnp.float32(jnp.inf)
    for k in range(K):
        m = jnp.min(d2, axis=1, keepdims=True)
        j = jnp.min(jnp.where(d2 == m, col, jnp.int32(N)), axis=1, keepdims=True)
        knn_ref[:, k : k + 1] = j
        if k + 1 < K:
            d2 = jnp.where(col == j, inf, d2)


def _knn(pxr, pyr, pzr, npx, npy, npz):
    grid = M // _KNN_R
    return pl.pallas_call(
        _knn_body,
        grid=(grid,),
        in_specs=[
            pl.BlockSpec((1, N), lambda i: (0, 0)),
            pl.BlockSpec((1, N), lambda i: (0, 0)),
            pl.BlockSpec((1, N), lambda i: (0, 0)),
            pl.BlockSpec((_KNN_R, 1), lambda i: (i, 0)),
            pl.BlockSpec((_KNN_R, 1), lambda i: (i, 0)),
            pl.BlockSpec((_KNN_R, 1), lambda i: (i, 0)),
        ],
        out_specs=pl.BlockSpec((_KNN_R, K), lambda i: (i, 0)),
        out_shape=jax.ShapeDtypeStruct((M, K), jnp.int32),
    )(pxr, pyr, pzr, npx, npy, npz)


_GYB = 2048
_GYG = N // _GYB


def _tstats_body(x_ref, w2_ref, out_ref, acc_ref):
    i = pl.program_id(0)

    @pl.when(i == 0)
    def _():
        acc_ref[...] = jnp.zeros_like(acc_ref)

    t = jnp.dot(x_ref[...], w2_ref[...], preferred_element_type=jnp.float32)
    acc_ref[0:1, :] += jnp.sum(t, axis=0, keepdims=True)
    acc_ref[1:2, :] += jnp.sum(t * t, axis=0, keepdims=True)

    @pl.when(i == _GYG - 1)
    def _():
        out_ref[...] = acc_ref[...]


def _tstats(x, w2):
    return pl.pallas_call(
        _tstats_body,
        grid=(_GYG,),
        in_specs=[
            pl.BlockSpec((_GYB, CIN), lambda i: (i, 0)),
            pl.BlockSpec((CIN, COUT), lambda i: (0, 0)),
        ],
        out_specs=pl.BlockSpec((2, COUT), lambda i: (0, 0)),
        out_shape=jax.ShapeDtypeStruct((2, COUT), jnp.float32),
        scratch_shapes=[pltpu.VMEM((2, COUT), jnp.float32)],
    )(x, w2)


def _gy_body(pt_ref, x_ref, ws1_ref, w2_ref, g2_ref, b2_ref, st_ref, gy_ref):
    x = x_ref[...]
    ws1 = ws1_ref[...]
    g = lax.dot_general(pt_ref[...], ws1[0:3, :], (((0,), (0,)), ((), ())),
                        preferred_element_type=jnp.float32)
    g = g + jnp.dot(x, ws1[3:, :], preferred_element_type=jnp.float32)
    gy_ref[:, 0:CIN] = g
    inv_n = jnp.float32(1.0 / N)
    mu = st_ref[0:1, :] * inv_n
    var = st_ref[1:2, :] * inv_n - mu * mu
    t = jnp.dot(x, w2_ref[...], preferred_element_type=jnp.float32)
    y = g2_ref[...] * (t - mu) / jnp.sqrt(var + EPS) + b2_ref[...]
    gy_ref[:, CIN:CIN + COUT] = jnp.maximum(y, 0.0)
    gy_ref[:, CIN + COUT:] = jnp.zeros((_GYB, 64), jnp.float32)


def _gy(pt, x, ws1, w2, g2, b2, st):
    return pl.pallas_call(
        _gy_body,
        grid=(_GYG,),
        in_specs=[
            pl.BlockSpec((3, _GYB), lambda i: (0, i)),
            pl.BlockSpec((_GYB, CIN), lambda i: (i, 0)),
            pl.BlockSpec((67, CIN), lambda i: (0, 0)),
            pl.BlockSpec((CIN, COUT), lambda i: (0, 0)),
            pl.BlockSpec((1, COUT), lambda i: (0, 0)),
            pl.BlockSpec((1, COUT), lambda i: (0, 0)),
            pl.BlockSpec((2, COUT), lambda i: (0, 0)),
        ],
        out_specs=pl.BlockSpec((_GYB, _D), lambda i: (i, 0)),
        out_shape=jax.ShapeDtypeStruct((N, _D), jnp.float32),
    )(pt, x, ws1, w2, g2, b2, st)


_D = 256
_CH = 128
_NW = 32
_CPW = (M * K) // _CH // _NW


def _sc_gather_body(gy_hbm, idx_hbm, out_hbm, idx_v, rows_v, sem):
    c = lax.axis_index("c")
    s = lax.axis_index("s")
    wid = s * 2 + c

    def chunk(j, carry):
        r = wid * _CPW + j
        pltpu.sync_copy(idx_hbm.at[r], idx_v)
        pltpu.async_copy(gy_hbm.at[idx_v], rows_v, sem).wait()
        pltpu.sync_copy(rows_v, out_hbm.at[pl.ds(r * _CH, _CH)])
        return carry

    lax.fori_loop(0, _CPW, chunk, 0)


def _sc_gather(gy, idx2d):
    mesh = plsc.VectorSubcoreMesh(core_axis_name="c", subcore_axis_name="s")
    fn = functools.partial(
        pl.kernel,
        out_type=jax.ShapeDtypeStruct((M * K, _D), jnp.float32),
        mesh=mesh,
        scratch_types=[
            pltpu.VMEM((_CH,), jnp.int32),
            pltpu.VMEM((_CH, _D), jnp.float32),
            pltpu.SemaphoreType.DMA,
        ],
    )(_sc_gather_body)
    return fn(gy, idx2d)


_SB = 256
_SG = M // _SB


def _stats_body(gk_ref, npx_ref, npy_ref, npz_ref, wp_ref, out_ref, acc_ref):
    i = pl.program_id(0)

    @pl.when(i == 0)
    def _():
        acc_ref[...] = jnp.zeros_like(acc_ref)

    wp = wp_ref[...]
    cb = (npx_ref[...] * wp[0:1, :] + npy_ref[...] * wp[1:2, :]
          + npz_ref[...] * wp[2:3, :])
    h = gk_ref[...].reshape(_SB, K, CIN) - cb[:, None, :]
    acc_ref[0:1, :] += jnp.sum(h, axis=(0, 1), keepdims=True).reshape(1, CIN)
    acc_ref[1:2, :] += jnp.sum(h * h, axis=(0, 1), keepdims=True).reshape(1, CIN)

    @pl.when(i == _SG - 1)
    def _():
        out_ref[...] = acc_ref[...]


def _stats(gyk, npx, npy, npz, wp):
    return pl.pallas_call(
        _stats_body,
        grid=(_SG,),
        in_specs=[
            pl.BlockSpec((_SB * K, CIN), lambda i: (i, 0)),
            pl.BlockSpec((_SB, 1), lambda i: (i, 0)),
            pl.BlockSpec((_SB, 1), lambda i: (i, 0)),
            pl.BlockSpec((_SB, 1), lambda i: (i, 0)),
            pl.BlockSpec((3, CIN), lambda i: (0, 0)),
        ],
        out_specs=pl.BlockSpec((2, CIN), lambda i: (0, 0)),
        out_shape=jax.ShapeDtypeStruct((2, CIN), jnp.float32),
        scratch_shapes=[pltpu.VMEM((2, CIN), jnp.float32)],
    )(gyk, npx, npy, npz, wp)


def _apply_body(st_ref, gyk_ref, npx_ref, npy_ref, npz_ref, wp_ref, gs_ref,
                bs_ref, w2s_ref, bs2_ref, out_ref):
    inv_n = jnp.float32(1.0 / (M * K))
    mu = st_ref[0:1, :] * inv_n
    var = st_ref[1:2, :] * inv_n - mu * mu
    wp = wp_ref[...]
    cb = (npx_ref[...] * wp[0:1, :] + npy_ref[...] * wp[1:2, :]
          + npz_ref[...] * wp[2:3, :])
    gy = gyk_ref[...]
    h = gy[:, 0:CIN].reshape(_SB, K, CIN) - cb[:, None, :]
    mu3 = mu.reshape(1, 1, CIN)
    den3 = jnp.sqrt(var + EPS).reshape(1, 1, CIN)
    hn = gs_ref[...].reshape(1, 1, CIN) * (h - mu3) / den3 \
        + bs_ref[...].reshape(1, 1, CIN)
    hn = jnp.maximum(hn, 0.0)
    s = jnp.sum(hn * w2s_ref[...].reshape(1, 1, CIN), axis=2) + bs2_ref[0, 0]
    sm = jnp.max(s, axis=1, keepdims=True)
    e = jnp.exp(s - sm)
    prob = e / jnp.sum(e, axis=1, keepdims=True)
    y3 = gy[:, CIN:CIN + COUT].reshape(_SB, K, COUT)
    out_ref[...] = jnp.sum(y3 * prob[:, :, None], axis=1)


def _apply(st, gyk, npx, npy, npz, wp, gs, bs, w2s, bs2):
    return pl.pallas_call(
        _apply_body,
        grid=(_SG,),
        in_specs=[
            pl.BlockSpec((2, CIN), lambda i: (0, 0)),
            pl.BlockSpec((_SB * K, _D), lambda i: (i, 0)),
            pl.BlockSpec((_SB, 1), lambda i: (i, 0)),
            pl.BlockSpec((_SB, 1), lambda i: (i, 0)),
            pl.BlockSpec((_SB, 1), lambda i: (i, 0)),
            pl.BlockSpec((3, CIN), lambda i: (0, 0)),
            pl.BlockSpec((1, CIN), lambda i: (0, 0)),
            pl.BlockSpec((1, CIN), lambda i: (0, 0)),
            pl.BlockSpec((1, CIN), lambda i: (0, 0)),
            pl.BlockSpec((1, 1), lambda i: (0, 0)),
        ],
        out_specs=pl.BlockSpec((_SB, COUT), lambda i: (i, 0)),
        out_shape=jax.ShapeDtypeStruct((M, COUT), jnp.float32),
    )(st, gyk, npx, npy, npz, wp, gs, bs, w2s, bs2)


def kernel(p, x, o, Ws1, gs, bs, Ws2, bs2, W2, g2, b2):
    px = p[:, 0]
    py = p[:, 1]
    pz = p[:, 2]
    npx, npy, npz = _fps(px.reshape(128, 128), py.reshape(128, 128),
                         pz.reshape(128, 128))
    knn = _knn(px.reshape(1, N), py.reshape(1, N), pz.reshape(1, N),
               npx, npy, npz)
    tst = _tstats(x, W2)
    gy = _gy(p.T, x, Ws1, W2, g2.reshape(1, COUT), b2.reshape(1, COUT), tst)
    gyk = _sc_gather(gy, knn.reshape((M * K) // _CH, _CH))
    wp = Ws1[0:3, :]
    st = _stats(gyk[:, 0:CIN], npx, npy, npz, wp)
    y_out = _apply(st, gyk, npx, npy, npz, wp, gs.reshape(1, CIN),
                   bs.reshape(1, CIN), Ws2.reshape(1, CIN),
                   bs2.reshape(1, 1))
    n_p = jnp.concatenate([npx, npy, npz], axis=1)
    n_o = jnp.array([M], dtype=jnp.int32)
    return (n_p, y_out, n_o)


# speedup vs baseline: 14.0671x; 1.4023x over previous
import functools

import jax
import jax.numpy as jnp
from jax import lax
from jax.experimental import pallas as pl
from jax.experimental.pallas import tpu as pltpu
from jax.experimental.pallas import tpu_sc as plsc

N = 16384
CIN = 64
COUT = 128
M = 4096
K = 16
EPS = 1e-5


def _fps_body(px_ref, py_ref, pz_ref, npx_ref, npy_ref, npz_ref, dist_ref):
    dist_ref[...] = jnp.full((128, 128), 1e10, jnp.float32)
    npx_ref[0:1, :] = px_ref[0:1, 0:1]
    npy_ref[0:1, :] = py_ref[0:1, 0:1]
    npz_ref[0:1, :] = pz_ref[0:1, 0:1]

    def body(i, carry):
        lx, ly, lz = carry
        px = px_ref[...]
        py = py_ref[...]
        pz = pz_ref[...]
        dx = px - lx
        dy = py - ly
        dz = pz - lz
        d = (dx * dx + dz * dz) + dy * dy
        dist = jnp.minimum(dist_ref[...], d)
        dist_ref[...] = dist
        v, cx, cy, cz = dist, px, py, pz
        h = 64
        while h >= 1:
            ta = v[0:h, :] >= v[h : 2 * h, :]
            v = jnp.where(ta, v[0:h, :], v[h : 2 * h, :])
            cx = jnp.where(ta, cx[0:h, :], cx[h : 2 * h, :])
            cy = jnp.where(ta, cy[0:h, :], cy[h : 2 * h, :])
            cz = jnp.where(ta, cz[0:h, :], cz[h : 2 * h, :])
            h //= 2
        v = v.reshape(1, 128).T
        cx = cx.reshape(1, 128).T
        cy = cy.reshape(1, 128).T
        cz = cz.reshape(1, 128).T
        h = 64
        while h >= 1:
            ta = v[0:h, :] >= v[h : 2 * h, :]
            v = jnp.where(ta, v[0:h, :], v[h : 2 * h, :])
            cx = jnp.where(ta, cx[0:h, :], cx[h : 2 * h, :])
            cy = jnp.where(ta, cy[0:h, :], cy[h : 2 * h, :])
            cz = jnp.where(ta, cz[0:h, :], cz[h : 2 * h, :])
            h //= 2
        npx_ref[pl.ds(i, 1), :] = cx
        npy_ref[pl.ds(i, 1), :] = cy
        npz_ref[pl.ds(i, 1), :] = cz
        return (cx[0, 0], cy[0, 0], cz[0, 0])

    lax.fori_loop(1, M, body,
                  (px_ref[0, 0], py_ref[0, 0], pz_ref[0, 0]))


def _fps(px, py, pz):
    out = jax.ShapeDtypeStruct((M, 1), jnp.float32)
    return pl.pallas_call(
        _fps_body,
        out_shape=(out, out, out),
        scratch_shapes=[pltpu.VMEM((128, 128), jnp.float32)],
    )(px, py, pz)


_KNN_R = 128


def _knn_body(pxr_ref, pyr_ref, pzr_ref, npx_ref, npy_ref, npz_ref, knn_ref):
    px = pxr_ref[...]
    py = pyr_ref[...]
    pz = pzr_ref[...]
    dx = npx_ref[...] - px
    dy = npy_ref[...] - py
    dz = npz_ref[...] - pz
    d2 = dx * dx + dy * dy + dz * dz
    col = lax.broadcasted_iota(jnp.int32, (_KNN_R, N), 1)
    inf = jnp.float32(jnp.inf)
    for k in range(K):
        m = jnp.min(d2, axis=1, keepdims=True)
        j = jnp.min(jnp.where(d2 == m, col, jnp.int32(N)), axis=1, keepdims=True)
        knn_ref[:, k : k + 1] = j
        if k + 1 < K:
            d2 = jnp.where(col == j, inf, d2)


def _knn(pxr, pyr, pzr, npx, npy, npz):
    grid = M // _KNN_R
    return pl.pallas_call(
        _knn_body,
        grid=(grid,),
        in_specs=[
            pl.BlockSpec((1, N), lambda i: (0, 0)),
            pl.BlockSpec((1, N), lambda i: (0, 0)),
            pl.BlockSpec((1, N), lambda i: (0, 0)),
            pl.BlockSpec((_KNN_R, 1), lambda i: (i, 0)),
            pl.BlockSpec((_KNN_R, 1), lambda i: (i, 0)),
            pl.BlockSpec((_KNN_R, 1), lambda i: (i, 0)),
        ],
        out_specs=pl.BlockSpec((_KNN_R, K), lambda i: (i, 0)),
        out_shape=jax.ShapeDtypeStruct((M, K), jnp.int32),
    )(pxr, pyr, pzr, npx, npy, npz)


_GYB = 2048
_GYG = N // _GYB


def _tstats_body(x_ref, w2_ref, out_ref, acc_ref):
    i = pl.program_id(0)

    @pl.when(i == 0)
    def _():
        acc_ref[...] = jnp.zeros_like(acc_ref)

    t = jnp.dot(x_ref[...], w2_ref[...], preferred_element_type=jnp.float32)
    acc_ref[0:1, :] += jnp.sum(t, axis=0, keepdims=True)
    acc_ref[1:2, :] += jnp.sum(t * t, axis=0, keepdims=True)

    @pl.when(i == _GYG - 1)
    def _():
        out_ref[...] = acc_ref[...]


def _tstats(x, w2):
    return pl.pallas_call(
        _tstats_body,
        grid=(_GYG,),
        in_specs=[
            pl.BlockSpec((_GYB, CIN), lambda i: (i, 0)),
            pl.BlockSpec((CIN, COUT), lambda i: (0, 0)),
        ],
        out_specs=pl.BlockSpec((2, COUT), lambda i: (0, 0)),
        out_shape=jax.ShapeDtypeStruct((2, COUT), jnp.float32),
        scratch_shapes=[pltpu.VMEM((2, COUT), jnp.float32)],
    )(x, w2)


def _gy_body(pt_ref, x_ref, ws1_ref, w2_ref, g2_ref, b2_ref, st_ref, gy_ref):
    x = x_ref[...]
    ws1 = ws1_ref[...]
    g = lax.dot_general(pt_ref[...], ws1[0:3, :], (((0,), (0,)), ((), ())),
                        preferred_element_type=jnp.float32)
    g = g + jnp.dot(x, ws1[3:, :], preferred_element_type=jnp.float32)
    gy_ref[:, 0:CIN] = g
    inv_n = jnp.float32(1.0 / N)
    mu = st_ref[0:1, :] * inv_n
    var = st_ref[1:2, :] * inv_n - mu * mu
    t = jnp.dot(x, w2_ref[...], preferred_element_type=jnp.float32)
    y = g2_ref[...] * (t - mu) / jnp.sqrt(var + EPS) + b2_ref[...]
    gy_ref[:, CIN:CIN + COUT] = jnp.maximum(y, 0.0)
    gy_ref[:, CIN + COUT:] = jnp.zeros((_GYB, 64), jnp.float32)


def _gy(pt, x, ws1, w2, g2, b2, st):
    return pl.pallas_call(
        _gy_body,
        grid=(_GYG,),
        in_specs=[
            pl.BlockSpec((3, _GYB), lambda i: (0, i)),
            pl.BlockSpec((_GYB, CIN), lambda i: (i, 0)),
            pl.BlockSpec((67, CIN), lambda i: (0, 0)),
            pl.BlockSpec((CIN, COUT), lambda i: (0, 0)),
            pl.BlockSpec((1, COUT), lambda i: (0, 0)),
            pl.BlockSpec((1, COUT), lambda i: (0, 0)),
            pl.BlockSpec((2, COUT), lambda i: (0, 0)),
        ],
        out_specs=pl.BlockSpec((_GYB, _D), lambda i: (i, 0)),
        out_shape=jax.ShapeDtypeStruct((N, _D), jnp.float32),
    )(pt, x, ws1, w2, g2, b2, st)


_D = 256
_CH = 128
_NW = 32
_CPW = (M * K) // _CH // _NW


def _sc_gather_body(gy_hbm, idx_hbm, out_hbm, idx_v, rows_v, sem):
    c = lax.axis_index("c")
    s = lax.axis_index("s")
    wid = s * 2 + c

    def chunk(j, carry):
        r = wid * _CPW + j
        pltpu.sync_copy(idx_hbm.at[r], idx_v)
        pltpu.async_copy(gy_hbm.at[idx_v], rows_v, sem).wait()
        pltpu.sync_copy(rows_v, out_hbm.at[pl.ds(r * _CH, _CH)])
        return carry

    lax.fori_loop(0, _CPW, chunk, 0)


def _sc_gather(gy, idx2d):
    mesh = plsc.VectorSubcoreMesh(core_axis_name="c", subcore_axis_name="s")
    fn = functools.partial(
        pl.kernel,
        out_type=jax.ShapeDtypeStruct((M * K, _D), jnp.float32),
        mesh=mesh,
        scratch_types=[
            pltpu.VMEM((_CH,), jnp.int32),
            pltpu.VMEM((_CH, _D), jnp.float32),
            pltpu.SemaphoreType.DMA,
        ],
    )(_sc_gather_body)
    return fn(gy, idx2d)


_SB = 256
_SG = M // _SB


def _stats_body(gk_ref, npx_ref, npy_ref, npz_ref, wp_ref, out_ref, acc_ref):
    i = pl.program_id(0)

    @pl.when(i == 0)
    def _():
        acc_ref[...] = jnp.zeros_like(acc_ref)

    wp = wp_ref[...]
    cb = (npx_ref[...] * wp[0:1, :] + npy_ref[...] * wp[1:2, :]
          + npz_ref[...] * wp[2:3, :])
    h = gk_ref[...].reshape(_SB, K, CIN) - cb[:, None, :]
    acc_ref[0:1, :] += jnp.sum(h, axis=(0, 1), keepdims=True).reshape(1, CIN)
    acc_ref[1:2, :] += jnp.sum(h * h, axis=(0, 1), keepdims=True).reshape(1, CIN)

    @pl.when(i == _SG - 1)
    def _():
        out_ref[...] = acc_ref[...]


def _stats(gyk, npx, npy, npz, wp):
    return pl.pallas_call(
        _stats_body,
        grid=(_SG,),
        in_specs=[
            pl.BlockSpec((_SB * K, CIN), lambda i: (i, 0)),
            pl.BlockSpec((_SB, 1), lambda i: (i, 0)),
            pl.BlockSpec((_SB, 1), lambda i: (i, 0)),
            pl.BlockSpec((_SB, 1), lambda i: (i, 0)),
            pl.BlockSpec((3, CIN), lambda i: (0, 0)),
        ],
        out_specs=pl.BlockSpec((2, CIN), lambda i: (0, 0)),
        out_shape=jax.ShapeDtypeStruct((2, CIN), jnp.float32),
        scratch_shapes=[pltpu.VMEM((2, CIN), jnp.float32)],
    )(gyk, npx, npy, npz, wp)


def _apply_body(st_ref, gyk_ref, npx_ref, npy_ref, npz_ref, wp_ref, gs_ref,
                bs_ref, w2s_ref, bs2_ref, out_ref):
    inv_n = jnp.float32(1.0 / (M * K))
    mu = st_ref[0:1, :] * inv_n
    var = st_ref[1:2, :] * inv_n - mu * mu
    wp = wp_ref[...]
    cb = (npx_ref[...] * wp[0:1, :] + npy_ref[...] * wp[1:2, :]
          + npz_ref[...] * wp[2:3, :])
    gy = gyk_ref[...]
    h = gy[:, 0:CIN].reshape(_SB, K, CIN) - cb[:, None, :]
    mu3 = mu.reshape(1, 1, CIN)
    den3 = jnp.sqrt(var + EPS).reshape(1, 1, CIN)
    hn = gs_ref[...].reshape(1, 1, CIN) * (h - mu3) / den3 \
        + bs_ref[...].reshape(1, 1, CIN)
    hn = jnp.maximum(hn, 0.0)
    s = jnp.sum(hn * w2s_ref[...].reshape(1, 1, CIN), axis=2) + bs2_ref[0, 0]
    sm = jnp.max(s, axis=1, keepdims=True)
    e = jnp.exp(s - sm)
    prob = e / jnp.sum(e, axis=1, keepdims=True)
    y3 = gy[:, CIN:CIN + COUT].reshape(_SB, K, COUT)
    out_ref[...] = jnp.sum(y3 * prob[:, :, None], axis=1)


def _apply(st, gyk, npx, npy, npz, wp, gs, bs, w2s, bs2):
    return pl.pallas_call(
        _apply_body,
        grid=(_SG,),
        in_specs=[
            pl.BlockSpec((2, CIN), lambda i: (0, 0)),
            pl.BlockSpec((_SB * K, _D), lambda i: (i, 0)),
            pl.BlockSpec((_SB, 1), lambda i: (i, 0)),
            pl.BlockSpec((_SB, 1), lambda i: (i, 0)),
            pl.BlockSpec((_SB, 1), lambda i: (i, 0)),
            pl.BlockSpec((3, CIN), lambda i: (0, 0)),
            pl.BlockSpec((1, CIN), lambda i: (0, 0)),
            pl.BlockSpec((1, CIN), lambda i: (0, 0)),
            pl.BlockSpec((1, CIN), lambda i: (0, 0)),
            pl.BlockSpec((1, 1), lambda i: (0, 0)),
        ],
        out_specs=pl.BlockSpec((_SB, COUT), lambda i: (i, 0)),
        out_shape=jax.ShapeDtypeStruct((M, COUT), jnp.float32),
    )(st, gyk, npx, npy, npz, wp, gs, bs, w2s, bs2)


def kernel(p, x, o, Ws1, gs, bs, Ws2, bs2, W2, g2, b2):
    px = p[:, 0]
    py = p[:, 1]
    pz = p[:, 2]
    npx, npy, npz = _fps(px.reshape(128, 128).T, py.reshape(128, 128).T,
                         pz.reshape(128, 128).T)
    knn = _knn(px.reshape(1, N), py.reshape(1, N), pz.reshape(1, N),
               npx, npy, npz)
    tst = _tstats(x, W2)
    gy = _gy(p.T, x, Ws1, W2, g2.reshape(1, COUT), b2.reshape(1, COUT), tst)
    gyk = _sc_gather(gy, knn.reshape((M * K) // _CH, _CH))
    wp = Ws1[0:3, :]
    st = _stats(gyk[:, 0:CIN], npx, npy, npz, wp)
    y_out = _apply(st, gyk, npx, npy, npz, wp, gs.reshape(1, CIN),
                   bs.reshape(1, CIN), Ws2.reshape(1, CIN),
                   bs2.reshape(1, 1))
    n_p = jnp.concatenate([npx, npy, npz], axis=1)
    n_o = jnp.array([M], dtype=jnp.int32)
    return (n_p, y_out, n_o)
